# baseline (speedup 1.0000x reference)
import jax
import jax.numpy as jnp
from jax.experimental import pallas as pl
from jax.experimental.pallas import tpu as pltpu

_BM = 1536


def _matmul_kernel(enc_ref, cb_ref, out_ref):
    out_ref[...] = jnp.dot(
        enc_ref[...], cb_ref[...], preferred_element_type=jnp.float32
    )


def kernel(encodings, codebook):
    m, k = encodings.shape
    _, n = codebook.shape
    return pl.pallas_call(
        _matmul_kernel,
        grid=(m // _BM,),
        in_specs=[
            pl.BlockSpec((_BM, k), lambda i: (i, 0)),
            pl.BlockSpec((k, n), lambda i: (0, 0)),
        ],
        out_specs=pl.BlockSpec((_BM, n), lambda i: (i, 0)),
        out_shape=jax.ShapeDtypeStruct((m, n), jnp.float32),
        compiler_params=pltpu.CompilerParams(
            dimension_semantics=("arbitrary",),
        ),
    )(encodings, codebook)


# speedup vs baseline: 1.0309x; 1.0309x over previous
import jax
import jax.numpy as jnp
from jax.experimental import pallas as pl
from jax.experimental.pallas import tpu as pltpu

_BM = 2304


def _matmul_kernel(enc_ref, cb_ref, out_ref):
    out_ref[...] = jnp.dot(
        enc_ref[...], cb_ref[...], preferred_element_type=jnp.float32
    )


def kernel(encodings, codebook):
    m, k = encodings.shape
    _, n = codebook.shape
    return pl.pallas_call(
        _matmul_kernel,
        grid=(m // _BM,),
        in_specs=[
            pl.BlockSpec((_BM, k), lambda i: (i, 0)),
            pl.BlockSpec((k, n), lambda i: (0, 0)),
        ],
        out_specs=pl.BlockSpec((_BM, n), lambda i: (i, 0)),
        out_shape=jax.ShapeDtypeStruct((m, n), jnp.float32),
        compiler_params=pltpu.CompilerParams(
            dimension_semantics=("arbitrary",),
        ),
    )(encodings, codebook)
